# baseline (speedup 1.0000x reference)
import functools

import jax
import jax.numpy as jnp
from jax import lax
from jax.experimental import pallas as pl
from jax.experimental.pallas import tpu as pltpu
from jax.experimental.pallas import tpu_sc as plsc

_N = 10000
_E = 160000
_D_IN = 256
_D_HID = 128
_MLP_HID = 200
_N_CLS = 2

_NC = 2
_NS = 16
_NW = _NC * _NS
_CH = 128
_EW = _E // _NW
_DCH = 40
_NA = 10240
_ZC = 5
_WR_T = 10
_WR_R = _N // _WR_T
_NBUF = 4

_sc_mesh = plsc.VectorSubcoreMesh(core_axis_name="c", subcore_axis_name="s")


@functools.partial(
    pl.kernel,
    out_type=jax.ShapeDtypeStruct((_NC, _N, 16), jnp.float32),
    mesh=_sc_mesh,
    scratch_types=[
        pltpu.VMEM((_DCH, _CH), jnp.int32),
        pltpu.VMEM((_CH, 16), jnp.float32),
        pltpu.VMEM((_CH, 16), jnp.float32),
        pltpu.VMEM_SHARED((_NA, 16), jnp.float32),
    ],
)
def _deg_kernel(dst_hbm, out_hbm, didx_v, ones_v, zero_v, acc_sh):
    c = lax.axis_index("c")
    s = lax.axis_index("s")
    w = c * _NS + s

    def fill(i, _):
        ones_v[i, :] = jnp.full((16,), 1.0, jnp.float32)
        zero_v[i, :] = jnp.zeros((16,), jnp.float32)
        return _
    lax.fori_loop(0, _CH, fill, None)

    pltpu.sync_copy(dst_hbm.at[pl.ds(w * _DCH, _DCH)], didx_v)
    for r in range(_ZC):
        pltpu.sync_copy(zero_v, acc_sh.at[pl.ds((s * _ZC + r) * _CH, _CH)])
    plsc.subcore_barrier()

    def chunk(j, _):
        pltpu.sync_copy(ones_v, acc_sh.at[didx_v.at[j]], add=True)
        return _
    lax.fori_loop(0, _DCH, chunk, None)
    plsc.subcore_barrier()

    @pl.when(s < _WR_T)
    def _():
        pltpu.sync_copy(acc_sh.at[pl.ds(s * _WR_R, _WR_R)],
                        out_hbm.at[c, pl.ds(s * _WR_R, _WR_R)])


@functools.partial(
    pl.kernel,
    out_type=jax.ShapeDtypeStruct((_NC, _N, _D_HID), jnp.float32),
    mesh=_sc_mesh,
    scratch_types=[
        pltpu.VMEM((_DCH, _CH), jnp.int32),
        pltpu.VMEM((_DCH, _CH), jnp.int32),
        pltpu.VMEM((_CH, _D_HID), jnp.float32),
        pltpu.VMEM((_CH, _D_HID), jnp.float32),
        pltpu.VMEM_SHARED((_NA, _D_HID), jnp.float32),
        pltpu.SemaphoreType.DMA,
    ],
)
def _edge_agg_kernel(hn_hbm, src_hbm, dst_hbm, out_hbm,
                     sidx_v, didx_v, rows_v, zero_v, acc_sh, sem):
    c = lax.axis_index("c")
    s = lax.axis_index("s")
    w = c * _NS + s

    def zfill(i, _):
        for k in range(_D_HID // 16):
            zero_v[i, pl.ds(k * 16, 16)] = jnp.zeros((16,), jnp.float32)
        return _
    lax.fori_loop(0, _CH, zfill, None)

    pltpu.sync_copy(src_hbm.at[pl.ds(w * _DCH, _DCH)], sidx_v)
    pltpu.sync_copy(dst_hbm.at[pl.ds(w * _DCH, _DCH)], didx_v)
    for r in range(_ZC):
        pltpu.sync_copy(zero_v, acc_sh.at[pl.ds((s * _ZC + r) * _CH, _CH)])
    plsc.subcore_barrier()

    def chunk(j, _):
        pltpu.async_copy(hn_hbm.at[sidx_v.at[j]], rows_v, sem).wait()
        pltpu.sync_copy(rows_v, acc_sh.at[didx_v.at[j]], add=True)
        return _
    lax.fori_loop(0, _DCH, chunk, None)
    plsc.subcore_barrier()

    @pl.when(s < _WR_T)
    def _():
        pltpu.sync_copy(acc_sh.at[pl.ds(s * _WR_R, _WR_R)],
                        out_hbm.at[c, pl.ds(s * _WR_R, _WR_R)])


_BLK = 2000
_NBLK = _N // _BLK


def _tc_a_body(d0_ref, d1_ref, x_ref, w1_ref, norm_ref, hn1_ref):
    d = d0_ref[:, 0:1] + d1_ref[:, 0:1] + 1.0
    nm = lax.rsqrt(d)
    h = jnp.dot(x_ref[...], w1_ref[...], preferred_element_type=jnp.float32)
    norm_ref[...] = nm
    hn1_ref[...] = h * nm


def _tc_b_body(s0_ref, s1_ref, hn1_ref, norm_ref, b1_ref, w2_ref, hn2_ref):
    nm = norm_ref[...]
    agg = (s0_ref[...] + s1_ref[...] + hn1_ref[...]) * nm + b1_ref[...]
    o = jnp.maximum(agg, 0.0)
    h2 = jnp.dot(o, w2_ref[...], preferred_element_type=jnp.float32)
    hn2_ref[...] = h2 * nm


def _tc_c1_body(s0_ref, s1_ref, hn2_ref, norm_ref, b2_ref, wm1_ref, bm1_ref,
                z_ref, sum_ref, sq_ref):
    agg = (s0_ref[...] + s1_ref[...] + hn2_ref[...]) * norm_ref[...] + b2_ref[...]
    h = jnp.maximum(agg, 0.0)
    z = jnp.dot(h, wm1_ref[...], preferred_element_type=jnp.float32) + bm1_ref[...]
    z = jnp.maximum(z, 0.0)
    z_ref[...] = z
    sum_ref[0, :, :] = jnp.sum(z, axis=0, keepdims=True)
    sq_ref[0, :, :] = jnp.sum(z * z, axis=0, keepdims=True)


def _tc_c2_body(z_ref, sum_ref, sq_ref, g_ref, bt_ref, wm2_ref, bm2_ref, out_ref):
    mean = jnp.sum(sum_ref[:, 0, :], axis=0, keepdims=True) * (1.0 / _N)
    var = jnp.sum(sq_ref[:, 0, :], axis=0, keepdims=True) * (1.0 / _N) - mean * mean
    zn = (z_ref[...] - mean) * lax.rsqrt(var + 1e-5) * g_ref[...] + bt_ref[...]
    out_ref[...] = (
        jnp.dot(zn, wm2_ref[...], preferred_element_type=jnp.float32)
        + bm2_ref[...]
    )


def _row_spec(width):
    return pl.BlockSpec((_BLK, width), lambda i: (i, 0))


def _full_spec(shape):
    return pl.BlockSpec(shape, lambda i: tuple(0 for _ in shape))


def _tc_a(d0, d1, x, w1):
    return pl.pallas_call(
        _tc_a_body,
        grid=(_NBLK,),
        in_specs=[_row_spec(16), _row_spec(16), _row_spec(_D_IN),
                  _full_spec((_D_IN, _D_HID))],
        out_specs=[_row_spec(1), _row_spec(_D_HID)],
        out_shape=[jax.ShapeDtypeStruct((_N, 1), jnp.float32),
                   jax.ShapeDtypeStruct((_NA, _D_HID), jnp.float32)],
    )(d0, d1, x, w1)


def _tc_b(s0, s1, hn1, norm, b1, w2):
    return pl.pallas_call(
        _tc_b_body,
        grid=(_NBLK,),
        in_specs=[_row_spec(_D_HID), _row_spec(_D_HID), _row_spec(_D_HID),
                  _row_spec(1), _full_spec((1, _D_HID)),
                  _full_spec((_D_HID, _D_HID))],
        out_specs=_row_spec(_D_HID),
        out_shape=jax.ShapeDtypeStruct((_NA, _D_HID), jnp.float32),
    )(s0, s1, hn1, norm, b1, w2)


def _tc_c1(s0, s1, hn2, norm, b2, wm1, bm1):
    return pl.pallas_call(
        _tc_c1_body,
        grid=(_NBLK,),
        in_specs=[_row_spec(_D_HID), _row_spec(_D_HID), _row_spec(_D_HID),
                  _row_spec(1), _full_spec((1, _D_HID)),
                  _full_spec((_D_HID, _MLP_HID)), _full_spec((1, _MLP_HID))],
        out_specs=[_row_spec(_MLP_HID),
                   pl.BlockSpec((1, 1, _MLP_HID), lambda i: (i, 0, 0)),
                   pl.BlockSpec((1, 1, _MLP_HID), lambda i: (i, 0, 0))],
        out_shape=[jax.ShapeDtypeStruct((_N, _MLP_HID), jnp.float32),
                   jax.ShapeDtypeStruct((_NBLK, 1, _MLP_HID), jnp.float32),
                   jax.ShapeDtypeStruct((_NBLK, 1, _MLP_HID), jnp.float32)],
    )(s0, s1, hn2, norm, b2, wm1, bm1)


def _tc_c2(z, sm, sq, gamma, beta, wm2, bm2):
    return pl.pallas_call(
        _tc_c2_body,
        grid=(_NBLK,),
        in_specs=[_row_spec(_MLP_HID), _full_spec((_NBLK, 1, _MLP_HID)),
                  _full_spec((_NBLK, 1, _MLP_HID)), _full_spec((1, _MLP_HID)),
                  _full_spec((1, _MLP_HID)), _full_spec((_MLP_HID, _N_CLS)),
                  _full_spec((1, _N_CLS))],
        out_specs=_row_spec(_N_CLS),
        out_shape=jax.ShapeDtypeStruct((_N, _N_CLS), jnp.float32),
    )(z, sm, sq, gamma, beta, wm2, bm2)


def _pad_edges(idx):
    w = idx.reshape(_NW, _EW)
    pad = jnp.full((_NW, _DCH * _CH - _EW), _N, jnp.int32)
    return jnp.concatenate([w, pad], axis=1).reshape(_NW * _DCH, _CH)


def kernel(features, edge_index, W1, b1, W2, b2, Wm1, bm1, gamma, beta, Wm2, bm2):
    src = _pad_edges(edge_index[0])
    dst = _pad_edges(edge_index[1])

    deg = _deg_kernel(dst)
    norm, hn1 = _tc_a(deg[0], deg[1], features, W1)

    s1 = _edge_agg_kernel(hn1, src, dst)
    hn2 = _tc_b(s1[0], s1[1], hn1, norm, b1.reshape(1, -1), W2)

    s2 = _edge_agg_kernel(hn2, src, dst)
    z, sm, sq = _tc_c1(s2[0], s2[1], hn2, norm, b2.reshape(1, -1), Wm1,
                       bm1.reshape(1, -1))
    return _tc_c2(z, sm, sq, gamma.reshape(1, -1), beta.reshape(1, -1), Wm2,
                  bm2.reshape(1, -1))


# speedup vs baseline: 1.0014x; 1.0014x over previous
import functools

import jax
import jax.numpy as jnp
from jax import lax
from jax.experimental import pallas as pl
from jax.experimental.pallas import tpu as pltpu
from jax.experimental.pallas import tpu_sc as plsc

_N = 10000
_E = 160000
_D_IN = 256
_D_HID = 128
_MLP_HID = 200
_N_CLS = 2

_NC = 2
_NS = 16
_NW = _NC * _NS
_CH = 128
_EW = _E // _NW
_DCH = 40
_NA = 10240
_ZC = 5
_WR_T = 10
_WR_R = _N // _WR_T
_NBUF = 4

_sc_mesh = plsc.VectorSubcoreMesh(core_axis_name="c", subcore_axis_name="s")


@functools.partial(
    pl.kernel,
    out_type=jax.ShapeDtypeStruct((_NC, _N, 16), jnp.float32),
    mesh=_sc_mesh,
    scratch_types=[
        pltpu.VMEM((_DCH, _CH), jnp.int32),
        pltpu.VMEM((_CH, 16), jnp.float32),
        pltpu.VMEM((_CH, 16), jnp.float32),
        pltpu.VMEM_SHARED((_NA, 16), jnp.float32),
    ],
)
def _deg_kernel(dst_hbm, out_hbm, didx_v, ones_v, zero_v, acc_sh):
    c = lax.axis_index("c")
    s = lax.axis_index("s")
    w = c * _NS + s

    def fill(i, _):
        ones_v[i, :] = jnp.full((16,), 1.0, jnp.float32)
        zero_v[i, :] = jnp.zeros((16,), jnp.float32)
        return _
    lax.fori_loop(0, _CH, fill, None)

    pltpu.sync_copy(dst_hbm.at[pl.ds(w * _DCH, _DCH)], didx_v)
    for r in range(_ZC):
        pltpu.sync_copy(zero_v, acc_sh.at[pl.ds((s * _ZC + r) * _CH, _CH)])
    plsc.subcore_barrier()

    def chunk(j, _):
        pltpu.sync_copy(ones_v, acc_sh.at[didx_v.at[j]], add=True)
        return _
    lax.fori_loop(0, _DCH, chunk, None)
    plsc.subcore_barrier()

    @pl.when(s < _WR_T)
    def _():
        pltpu.sync_copy(acc_sh.at[pl.ds(s * _WR_R, _WR_R)],
                        out_hbm.at[c, pl.ds(s * _WR_R, _WR_R)])


@functools.partial(
    pl.kernel,
    out_type=jax.ShapeDtypeStruct((_NC, _N, _D_HID), jnp.float32),
    mesh=_sc_mesh,
    scratch_types=[
        pltpu.VMEM((_DCH, _CH), jnp.int32),
        pltpu.VMEM((_DCH, _CH), jnp.int32),
        pltpu.VMEM((_CH, _D_HID), jnp.float32),
        pltpu.VMEM((_CH, _D_HID), jnp.float32),
        pltpu.VMEM_SHARED((_NA, _D_HID), jnp.float32),
        pltpu.SemaphoreType.DMA,
    ],
)
def _edge_agg_kernel(hn_hbm, src_hbm, dst_hbm, out_hbm,
                     sidx_v, didx_v, rows_v, zero_v, acc_sh, sem):
    c = lax.axis_index("c")
    s = lax.axis_index("s")
    w = c * _NS + s

    def zfill(i, _):
        for k in range(_D_HID // 16):
            zero_v[i, pl.ds(k * 16, 16)] = jnp.zeros((16,), jnp.float32)
        return _
    lax.fori_loop(0, _CH, zfill, None)

    pltpu.sync_copy(src_hbm.at[pl.ds(w * _DCH, _DCH)], sidx_v)
    pltpu.sync_copy(dst_hbm.at[pl.ds(w * _DCH, _DCH)], didx_v)
    for r in range(_ZC):
        pltpu.sync_copy(zero_v, acc_sh.at[pl.ds((s * _ZC + r) * _CH, _CH)])
    plsc.subcore_barrier()

    def chunk(j, _):
        pltpu.async_copy(hn_hbm.at[sidx_v.at[j]], rows_v, sem).wait()
        pltpu.sync_copy(rows_v, acc_sh.at[didx_v.at[j]], add=True)
        return _
    lax.fori_loop(0, _DCH, chunk, None)
    plsc.subcore_barrier()

    @pl.when(s < _WR_T)
    def _():
        pltpu.sync_copy(acc_sh.at[pl.ds(s * _WR_R, _WR_R)],
                        out_hbm.at[c, pl.ds(s * _WR_R, _WR_R)])


_BLK = 2000
_NBLK = _N // _BLK


def _tc_m_body(x_ref, w1_ref, h_ref):
    h_ref[...] = jnp.dot(x_ref[...], w1_ref[...],
                         preferred_element_type=jnp.float32)


def _tc_a_body(d0_ref, d1_ref, h_ref, norm_ref, hn1_ref):
    d = d0_ref[:, 0:1] + d1_ref[:, 0:1] + 1.0
    nm = lax.rsqrt(d)
    norm_ref[...] = nm
    hn1_ref[...] = h_ref[...] * nm


def _tc_b_body(s0_ref, s1_ref, hn1_ref, norm_ref, b1_ref, w2_ref, hn2_ref):
    nm = norm_ref[...]
    agg = (s0_ref[...] + s1_ref[...] + hn1_ref[...]) * nm + b1_ref[...]
    o = jnp.maximum(agg, 0.0)
    h2 = jnp.dot(o, w2_ref[...], preferred_element_type=jnp.float32)
    hn2_ref[...] = h2 * nm


def _tc_c1_body(s0_ref, s1_ref, hn2_ref, norm_ref, b2_ref, wm1_ref, bm1_ref,
                z_ref, sum_ref, sq_ref):
    agg = (s0_ref[...] + s1_ref[...] + hn2_ref[...]) * norm_ref[...] + b2_ref[...]
    h = jnp.maximum(agg, 0.0)
    z = jnp.dot(h, wm1_ref[...], preferred_element_type=jnp.float32) + bm1_ref[...]
    z = jnp.maximum(z, 0.0)
    z_ref[...] = z
    sum_ref[0, :, :] = jnp.sum(z, axis=0, keepdims=True)
    sq_ref[0, :, :] = jnp.sum(z * z, axis=0, keepdims=True)


def _tc_c2_body(z_ref, sum_ref, sq_ref, g_ref, bt_ref, wm2_ref, bm2_ref, out_ref):
    mean = jnp.sum(sum_ref[:, 0, :], axis=0, keepdims=True) * (1.0 / _N)
    var = jnp.sum(sq_ref[:, 0, :], axis=0, keepdims=True) * (1.0 / _N) - mean * mean
    zn = (z_ref[...] - mean) * lax.rsqrt(var + 1e-5) * g_ref[...] + bt_ref[...]
    out_ref[...] = (
        jnp.dot(zn, wm2_ref[...], preferred_element_type=jnp.float32)
        + bm2_ref[...]
    )


def _row_spec(width):
    return pl.BlockSpec((_BLK, width), lambda i: (i, 0))


def _full_spec(shape):
    return pl.BlockSpec(shape, lambda i: tuple(0 for _ in shape))


def _tc_m(x, w1):
    return pl.pallas_call(
        _tc_m_body,
        grid=(_NBLK,),
        in_specs=[_row_spec(_D_IN), _full_spec((_D_IN, _D_HID))],
        out_specs=_row_spec(_D_HID),
        out_shape=jax.ShapeDtypeStruct((_N, _D_HID), jnp.float32),
    )(x, w1)


def _tc_a(d0, d1, h):
    return pl.pallas_call(
        _tc_a_body,
        grid=(_NBLK,),
        in_specs=[_row_spec(16), _row_spec(16), _row_spec(_D_HID)],
        out_specs=[_row_spec(1), _row_spec(_D_HID)],
        out_shape=[jax.ShapeDtypeStruct((_N, 1), jnp.float32),
                   jax.ShapeDtypeStruct((_NA, _D_HID), jnp.float32)],
    )(d0, d1, h)


def _tc_b(s0, s1, hn1, norm, b1, w2):
    return pl.pallas_call(
        _tc_b_body,
        grid=(_NBLK,),
        in_specs=[_row_spec(_D_HID), _row_spec(_D_HID), _row_spec(_D_HID),
                  _row_spec(1), _full_spec((1, _D_HID)),
                  _full_spec((_D_HID, _D_HID))],
        out_specs=_row_spec(_D_HID),
        out_shape=jax.ShapeDtypeStruct((_NA, _D_HID), jnp.float32),
    )(s0, s1, hn1, norm, b1, w2)


def _tc_c1(s0, s1, hn2, norm, b2, wm1, bm1):
    return pl.pallas_call(
        _tc_c1_body,
        grid=(_NBLK,),
        in_specs=[_row_spec(_D_HID), _row_spec(_D_HID), _row_spec(_D_HID),
                  _row_spec(1), _full_spec((1, _D_HID)),
                  _full_spec((_D_HID, _MLP_HID)), _full_spec((1, _MLP_HID))],
        out_specs=[_row_spec(_MLP_HID),
                   pl.BlockSpec((1, 1, _MLP_HID), lambda i: (i, 0, 0)),
                   pl.BlockSpec((1, 1, _MLP_HID), lambda i: (i, 0, 0))],
        out_shape=[jax.ShapeDtypeStruct((_N, _MLP_HID), jnp.float32),
                   jax.ShapeDtypeStruct((_NBLK, 1, _MLP_HID), jnp.float32),
                   jax.ShapeDtypeStruct((_NBLK, 1, _MLP_HID), jnp.float32)],
    )(s0, s1, hn2, norm, b2, wm1, bm1)


def _tc_c2(z, sm, sq, gamma, beta, wm2, bm2):
    return pl.pallas_call(
        _tc_c2_body,
        grid=(_NBLK,),
        in_specs=[_row_spec(_MLP_HID), _full_spec((_NBLK, 1, _MLP_HID)),
                  _full_spec((_NBLK, 1, _MLP_HID)), _full_spec((1, _MLP_HID)),
                  _full_spec((1, _MLP_HID)), _full_spec((_MLP_HID, _N_CLS)),
                  _full_spec((1, _N_CLS))],
        out_specs=_row_spec(_N_CLS),
        out_shape=jax.ShapeDtypeStruct((_N, _N_CLS), jnp.float32),
    )(z, sm, sq, gamma, beta, wm2, bm2)


def _pad_edges(idx):
    w = idx.reshape(_NW, _EW)
    pad = jnp.full((_NW, _DCH * _CH - _EW), _N, jnp.int32)
    return jnp.concatenate([w, pad], axis=1).reshape(_NW * _DCH, _CH)


def kernel(features, edge_index, W1, b1, W2, b2, Wm1, bm1, gamma, beta, Wm2, bm2):
    src = _pad_edges(edge_index[0])
    dst = _pad_edges(edge_index[1])

    deg = _deg_kernel(dst)
    h1 = _tc_m(features, W1)
    norm, hn1 = _tc_a(deg[0], deg[1], h1)

    s1 = _edge_agg_kernel(hn1, src, dst)
    hn2 = _tc_b(s1[0], s1[1], hn1, norm, b1.reshape(1, -1), W2)

    s2 = _edge_agg_kernel(hn2, src, dst)
    z, sm, sq = _tc_c1(s2[0], s2[1], hn2, norm, b2.reshape(1, -1), Wm1,
                       bm1.reshape(1, -1))
    return _tc_c2(z, sm, sq, gamma.reshape(1, -1), beta.reshape(1, -1), Wm2,
                  bm2.reshape(1, -1))
